# Initial kernel scaffold; baseline (speedup 1.0000x reference)
import jax
import jax.numpy as jnp
from jax.experimental import pallas as pl


def kernel(token_ids, weight):
    raise NotImplementedError("write your pallas kernel here")


# speedup vs baseline: 1.8573x; 1.8573x over previous
import functools

import jax
import jax.numpy as jnp
from jax import lax
from jax.experimental import pallas as pl
from jax.experimental.pallas import tpu as pltpu
from jax.experimental.pallas import tpu_sc as plsc

NUM_CORES = 2
NUM_SUBCORES = 16
NW = NUM_CORES * NUM_SUBCORES
CHUNK = 128
NBUF = 8


@functools.lru_cache(maxsize=None)
def _make_gather(total: int, d: int):
    per_w = total // NW
    nchunk = per_w // CHUNK
    assert per_w * NW == total and nchunk * CHUNK == per_w and nchunk % NBUF == 0

    mesh = plsc.VectorSubcoreMesh(
        core_axis_name="c", subcore_axis_name="s",
        num_cores=NUM_CORES, num_subcores=NUM_SUBCORES)

    @functools.partial(
        pl.kernel,
        out_type=jax.ShapeDtypeStruct((NW, nchunk, CHUNK, d), jnp.float32),
        mesh=mesh,
        scratch_types=[
            pltpu.VMEM((nchunk, CHUNK), jnp.int32),
            pltpu.VMEM((NBUF, CHUNK, d), jnp.float32),
            pltpu.SemaphoreType.DMA,
            pltpu.SemaphoreType.DMA,
        ],
        compiler_params=pltpu.CompilerParams(use_tc_tiling_on_sc=False),
    )
    def gather_kernel(idx_hbm, table_hbm, out_hbm, idx_v, rows_v, sem_in, sem_out):
        wid = lax.axis_index("s") * NUM_CORES + lax.axis_index("c")
        pltpu.sync_copy(idx_hbm.at[wid], idx_v)

        @pl.loop(0, nchunk, step=NBUF)
        def _group(g):
            for b in range(NBUF):
                pltpu.make_async_copy(
                    table_hbm.at[idx_v.at[g + b]], rows_v.at[b], sem_in).start()
            for b in range(NBUF):
                pltpu.make_async_copy(
                    table_hbm.at[idx_v.at[g + b]], rows_v.at[b], sem_in).wait()
            for b in range(NBUF):
                pltpu.make_async_copy(
                    rows_v.at[b], out_hbm.at[wid, g + b], sem_out).start()
            for b in range(NBUF):
                pltpu.make_async_copy(
                    rows_v.at[b], out_hbm.at[wid, g + b], sem_out).wait()

    return gather_kernel


def kernel(token_ids, weight):
    batch, hist = token_ids.shape
    d = weight.shape[1]
    total = batch * hist
    idx = token_ids.astype(jnp.int32).reshape(NW, total // NW // CHUNK, CHUNK)
    out = _make_gather(total, d)(idx, weight)
    return out.reshape(batch, hist, d)


# speedup vs baseline: 1.8744x; 1.0092x over previous
import functools

import jax
import jax.numpy as jnp
from jax import lax
from jax.experimental import pallas as pl
from jax.experimental.pallas import tpu as pltpu
from jax.experimental.pallas import tpu_sc as plsc

NUM_CORES = 2
NUM_SUBCORES = 16
NW = NUM_CORES * NUM_SUBCORES
CHUNK = 128
K = 5


@functools.lru_cache(maxsize=None)
def _make_gather(total: int, d: int):
    per_w = total // NW
    nchunk = per_w // CHUNK
    assert per_w * NW == total and nchunk * CHUNK == per_w
    assert nchunk % (2 * K) == 0 and nchunk >= 4 * K

    mesh = plsc.VectorSubcoreMesh(
        core_axis_name="c", subcore_axis_name="s",
        num_cores=NUM_CORES, num_subcores=NUM_SUBCORES)

    @functools.partial(
        pl.kernel,
        out_type=jax.ShapeDtypeStruct((NW, nchunk, CHUNK, d), jnp.float32),
        mesh=mesh,
        scratch_types=[
            pltpu.VMEM((nchunk, CHUNK), jnp.int32),
            pltpu.VMEM((2, K, CHUNK, d), jnp.float32),
            pltpu.SemaphoreType.DMA,
            pltpu.SemaphoreType.DMA,
            pltpu.SemaphoreType.DMA,
            pltpu.SemaphoreType.DMA,
        ],
        compiler_params=pltpu.CompilerParams(use_tc_tiling_on_sc=False),
    )
    def gather_kernel(idx_hbm, table_hbm, out_hbm, idx_v, rows_v,
                      sem_in0, sem_in1, sem_out0, sem_out1):
        wid = lax.axis_index("s") * NUM_CORES + lax.axis_index("c")
        pltpu.sync_copy(idx_hbm.at[wid], idx_v)
        sem_in = (sem_in0, sem_in1)
        sem_out = (sem_out0, sem_out1)

        def fire_gather(grp, base):
            for b in range(K):
                pltpu.make_async_copy(
                    table_hbm.at[idx_v.at[base + b]],
                    rows_v.at[grp, b], sem_in[grp]).start()

        def drain_gather(grp, base):
            for b in range(K):
                pltpu.make_async_copy(
                    table_hbm.at[idx_v.at[base + b]],
                    rows_v.at[grp, b], sem_in[grp]).wait()

        def fire_write(grp, base):
            for b in range(K):
                pltpu.make_async_copy(
                    rows_v.at[grp, b], out_hbm.at[wid, base + b],
                    sem_out[grp]).start()

        def drain_write(grp, base):
            for b in range(K):
                pltpu.make_async_copy(
                    rows_v.at[grp, b], out_hbm.at[wid, base + b],
                    sem_out[grp]).wait()

        def half(grp, base, first=False, last=False):
            oth = 1 - grp
            if not first:
                drain_write(oth, base - K)
            if not last:
                fire_gather(oth, base + K)
            drain_gather(grp, base)
            fire_write(grp, base)

        fire_gather(0, 0)
        half(0, 0, first=True)
        half(1, K)

        @pl.loop(2 * K, nchunk - 2 * K, step=2 * K)
        def _steady(g):
            half(0, g)
            half(1, g + K)

        half(0, nchunk - 2 * K)
        half(1, nchunk - K, last=True)
        drain_write(1, nchunk - K)

    return gather_kernel


def kernel(token_ids, weight):
    batch, hist = token_ids.shape
    d = weight.shape[1]
    total = batch * hist
    idx = token_ids.astype(jnp.int32).reshape(NW, total // NW // CHUNK, CHUNK)
    out = _make_gather(total, d)(idx, weight)
    return out.reshape(batch, hist, d)
